# Initial kernel scaffold; baseline (speedup 1.0000x reference)
import jax
import jax.numpy as jnp
from jax.experimental import pallas as pl


def kernel(x, W1, b1, W2, b2, W3, b3, W4, b4):
    raise NotImplementedError("write your pallas kernel here")


# speedup vs baseline: 16.4417x; 16.4417x over previous
import functools

import jax
import jax.numpy as jnp
from jax.experimental import pallas as pl
from jax.experimental.pallas import tpu as pltpu


def _mm(a, w):
    return jax.lax.dot_general(
        a.astype(jnp.bfloat16), w,
        (((1,), (0,)), ((), ())),
        preferred_element_type=jnp.float32,
    )


def _fused_kernel(x_ref, w1_ref, b1_ref, w2_ref, b2_ref, w3_ref, b3_ref,
                  w4_ref, b4_ref, out_ref, *, k):
    x = x_ref[...]
    h1 = jnp.tanh(_mm(x, w1_ref[...]) + b1_ref[...])
    l2 = _mm(h1, w2_ref[...]) + b2_ref[...]
    m = jnp.max(l2, axis=1, keepdims=True)
    e = jnp.exp(l2 - m)
    p = e / jnp.sum(e, axis=1, keepdims=True)
    h3 = jnp.tanh(_mm(p, w3_ref[...]) + b3_ref[...])
    sep = _mm(h3, w4_ref[...]) + b4_ref[...]

    u = jax.lax.bitcast_convert_type(sep, jnp.uint32)
    neg = u >= jnp.uint32(0x80000000)
    key = jnp.where(neg, ~u, u | jnp.uint32(0x80000000))

    def body(i, kth):
        bit = jnp.left_shift(jnp.uint32(1), jnp.uint32(31) - i.astype(jnp.uint32))
        trial = kth | bit
        cnt = jnp.sum((key >= trial).astype(jnp.int32), axis=1, keepdims=True)
        return jnp.where(cnt >= k, trial, kth)

    kth = jax.lax.fori_loop(
        0, 32, body, jnp.zeros((sep.shape[0], 1), jnp.uint32))
    out_ref[...] = jnp.where(key >= kth, sep, 0.0)


@jax.jit
def kernel(x, W1, b1, W2, b2, W3, b3, W4, b4):
    n_tok, d_in = x.shape
    d_h = W1.shape[1]
    d_out = W4.shape[1]
    k = max(1, int(d_out * 0.1))

    blk = 256 if n_tok % 256 == 0 else n_tok
    grid = (n_tok // blk,)

    wspec = lambda shape: pl.BlockSpec(shape, lambda i: (0, 0))
    out = pl.pallas_call(
        functools.partial(_fused_kernel, k=k),
        grid=grid,
        in_specs=[
            pl.BlockSpec((blk, d_in), lambda i: (i, 0)),
            wspec((d_in, d_h)),
            wspec((1, d_h)),
            wspec((d_h, d_h)),
            wspec((1, d_h)),
            wspec((d_h, d_h)),
            wspec((1, d_h)),
            wspec((d_h, d_out)),
            wspec((1, d_out)),
        ],
        out_specs=pl.BlockSpec((blk, d_out), lambda i: (i, 0)),
        out_shape=jax.ShapeDtypeStruct((n_tok, d_out), jnp.float32),
        compiler_params=pltpu.CompilerParams(
            dimension_semantics=("parallel",),
        ),
    )(
        x.astype(jnp.bfloat16),
        W1.astype(jnp.bfloat16), b1.reshape(1, d_h).astype(jnp.float32),
        W2.astype(jnp.bfloat16), b2.reshape(1, d_h).astype(jnp.float32),
        W3.astype(jnp.bfloat16), b3.reshape(1, d_h).astype(jnp.float32),
        W4.astype(jnp.bfloat16), b4.reshape(1, d_out).astype(jnp.float32),
    )
    return out
